# baseline (speedup 1.0000x reference)
import functools

import jax
import jax.numpy as jnp
from jax import lax
from jax.experimental import pallas as pl
from jax.experimental.pallas import tpu as pltpu
from jax.experimental.pallas import tpu_sc as plsc

NC, NS, L = 2, 16, 16
NW = NC * NS
NE, D = 32, 128
NV = 64
B, C = 1024, 200
N = B * C
TPW = N // NW
KT = 128
NCHUNK = TPW // KT
NB = 2
U = 16

_mesh = plsc.VectorSubcoreMesh(
    core_axis_name="c", subcore_axis_name="s", num_cores=NC, num_subcores=NS
)


@functools.partial(
    pl.kernel,
    out_type=jax.ShapeDtypeStruct((N, D), jnp.float32),
    mesh=_mesh,
    scratch_types=[
        pltpu.VMEM((NE, D), jnp.float32),
        pltpu.VMEM((NV, D), jnp.float32),
        pltpu.VMEM((TPW,), jnp.int32),
        [pltpu.VMEM((KT, D), jnp.float32)] * NB,
        [pltpu.SemaphoreType.DMA] * NB,
    ],
)
def _vme_kernel(in_hbm, emb_hbm, out_hbm, raw_v, table_v, idx_all, stage, osem):
    cid = lax.axis_index("c")
    sid = lax.axis_index("s")
    wid = sid * NC + cid
    base = wid * TPW

    pltpu.sync_copy(emb_hbm, raw_v)
    pltpu.sync_copy(in_hbm.at[pl.ds(base, TPW)], idx_all)

    def build_row(r, carry):
        m = 0.5 + 0.0625 * (r % 16).astype(jnp.float32)
        rsrc = r % NE
        for j in range(D // L):
            sl = pl.ds(j * L, L)
            table_v[r, sl] = raw_v[rsrc, sl] * m
        return carry

    lax.fori_loop(0, NV, build_row, 0)

    def w_copy(c, b):
        return pltpu.make_async_copy(
            stage[b], out_hbm.at[pl.ds(base + c * KT, KT)], osem[b]
        )

    def expand(c, b):
        def group(g, carry):
            t0 = g * U
            idxv = idx_all[pl.ds(c * KT + t0, U)]
            for u in range(U):
                row = idxv[u]
                for j in range(D // L):
                    sl = pl.ds(j * L, L)
                    stage[b][t0 + u, sl] = table_v[row, sl]
            return carry

        lax.fori_loop(0, KT // U, group, 0)

    def step(s, carry):
        for b in range(NB):
            c = s * NB + b

            @pl.when(c >= NB)
            def _drain():
                w_copy(c - NB, b).wait()

            expand(c, b)
            w_copy(c, b).start()
        return carry

    lax.fori_loop(0, NCHUNK // NB, step, 0)

    for b in range(NB):
        w_copy(NCHUNK - NB + b, b).wait()


def kernel(input_BC, raw_embed):
    out = _vme_kernel(input_BC.reshape(N), raw_embed)
    return out.reshape(B, C, D)


# speedup vs baseline: 1.0542x; 1.0542x over previous
import functools

import jax
import jax.numpy as jnp
from jax import lax
from jax.experimental import pallas as pl
from jax.experimental.pallas import tpu as pltpu
from jax.experimental.pallas import tpu_sc as plsc

NC, NS, L = 2, 16, 16
NW = NC * NS
NE, D = 32, 128
NV = 64
B, C = 1024, 200
N = B * C
TPW = N // NW
KT = 128
NCHUNK = TPW // KT
NB = 2
U = 16

_mesh = plsc.VectorSubcoreMesh(
    core_axis_name="c", subcore_axis_name="s", num_cores=NC, num_subcores=NS
)


@functools.partial(
    pl.kernel,
    out_type=jax.ShapeDtypeStruct((N, D), jnp.float32),
    mesh=_mesh,
    scratch_types=[
        pltpu.VMEM((NE, D), jnp.float32),
        pltpu.VMEM((NV, D), jnp.float32),
        pltpu.VMEM((TPW,), jnp.int32),
        [pltpu.VMEM((KT, D), jnp.float32)] * NB,
        [pltpu.SemaphoreType.DMA] * NB,
    ],
    compiler_params=pltpu.CompilerParams(needs_layout_passes=False),
)
def _vme_kernel(in_hbm, emb_hbm, out_hbm, raw_v, table_v, idx_all, stage, osem):
    cid = lax.axis_index("c")
    sid = lax.axis_index("s")
    wid = sid * NC + cid
    base = wid * TPW

    pltpu.sync_copy(emb_hbm, raw_v)
    pltpu.sync_copy(in_hbm.at[pl.ds(base, TPW)], idx_all)

    def build_row(r, carry):
        m = 0.5 + 0.0625 * (r % 16).astype(jnp.float32)
        rsrc = r % NE
        for j in range(D // L):
            sl = pl.ds(j * L, L)
            table_v[r, sl] = raw_v[rsrc, sl] * m
        return carry

    lax.fori_loop(0, NV, build_row, 0)

    def w_copy(c, b):
        return pltpu.make_async_copy(
            stage[b], out_hbm.at[pl.ds(base + c * KT, KT)], osem[b]
        )

    iot = lax.iota(jnp.int32, 16)
    iotj = [iot + j * L for j in range(D // L)]

    def expand(c, b):
        def group(g, carry):
            t0 = g * U
            idxv = idx_all[pl.ds(c * KT + t0, U)]
            for u in range(U):
                rowb = jnp.full((16,), idxv[u], jnp.int32)
                for j in range(D // L):
                    g16 = plsc.load_gather(table_v, [rowb, iotj[j]])
                    stage[b][t0 + u, pl.ds(j * L, L)] = g16
            return carry

        lax.fori_loop(0, KT // U, group, 0)

    def step(s, carry):
        for b in range(NB):
            c = s * NB + b

            @pl.when(c >= NB)
            def _drain():
                w_copy(c - NB, b).wait()

            expand(c, b)
            w_copy(c, b).start()
        return carry

    lax.fori_loop(0, NCHUNK // NB, step, 0)

    for b in range(NB):
        w_copy(NCHUNK - NB + b, b).wait()


def kernel(input_BC, raw_embed):
    out = _vme_kernel(input_BC.reshape(N), raw_embed)
    return out.reshape(B, C, D)


# speedup vs baseline: 1.6108x; 1.5280x over previous
import functools

import jax
import jax.numpy as jnp
from jax import lax
from jax.experimental import pallas as pl
from jax.experimental.pallas import tpu as pltpu
from jax.experimental.pallas import tpu_sc as plsc

NC, NS, L = 2, 16, 16
NW = NC * NS
NE, D = 32, 128
NV = 64
B, C = 1024, 200
N = B * C
TPW = N // NW
KT = 128
NCHUNK = TPW // KT
NB = 2
U = 16

_mesh = plsc.VectorSubcoreMesh(
    core_axis_name="c", subcore_axis_name="s", num_cores=NC, num_subcores=NS
)


@functools.partial(
    pl.kernel,
    out_type=jax.ShapeDtypeStruct((N, D), jnp.float32),
    mesh=_mesh,
    scratch_types=[
        pltpu.VMEM((NE, D), jnp.float32),
        pltpu.VMEM((NV, D), jnp.float32),
        pltpu.VMEM((TPW,), jnp.int32),
        [pltpu.VMEM((KT, D), jnp.float32)] * NB,
        [pltpu.SemaphoreType.DMA] * NB,
    ],
    compiler_params=pltpu.CompilerParams(needs_layout_passes=False),
)
def _vme_kernel(in_hbm, emb_hbm, out_hbm, raw_v, table_v, idx_all, stage, osem):
    cid = lax.axis_index("c")
    sid = lax.axis_index("s")
    wid = sid * NC + cid
    base = wid * TPW

    pltpu.sync_copy(emb_hbm, raw_v)
    pltpu.sync_copy(in_hbm.at[pl.ds(base, TPW)], idx_all)

    def build_row(r, carry):
        m = 0.5 + 0.0625 * (r % 16).astype(jnp.float32)
        rsrc = r % NE
        for j in range(D // L):
            sl = pl.ds(j * L, L)
            table_v[r, sl] = raw_v[rsrc, sl] * m
        return carry

    lax.fori_loop(0, NV, build_row, 0)

    def w_copy(c, b):
        return pltpu.make_async_copy(
            stage[b], out_hbm.at[pl.ds(base + c * KT, KT)], osem[b]
        )

    iot = lax.iota(jnp.int32, 16)
    iotj = [iot + j * L for j in range(D // L)]

    def expand(c, b):
        @plsc.parallel_loop(0, KT // U)
        def group(g):
            t0 = g * U
            idxv = idx_all[pl.ds(c * KT + t0, U)]
            for u in range(U):
                rowb = jnp.full((16,), idxv[u], jnp.int32)
                for j in range(D // L):
                    g16 = plsc.load_gather(table_v, [rowb, iotj[j]])
                    stage[b][t0 + u, pl.ds(j * L, L)] = g16

    def step(s, carry):
        for b in range(NB):
            c = s * NB + b

            @pl.when(c >= NB)
            def _drain():
                w_copy(c - NB, b).wait()

            expand(c, b)
            w_copy(c, b).start()
        return carry

    lax.fori_loop(0, NCHUNK // NB, step, 0)

    for b in range(NB):
        w_copy(NCHUNK - NB + b, b).wait()


def kernel(input_BC, raw_embed):
    out = _vme_kernel(input_BC.reshape(N), raw_embed)
    return out.reshape(B, C, D)
